# Initial kernel scaffold; baseline (speedup 1.0000x reference)
import jax
import jax.numpy as jnp
from jax.experimental import pallas as pl


def kernel(fp4_xyz, fp4_features, concatenate_features, Wq, bq, Wk, bk):
    raise NotImplementedError("write your pallas kernel here")


# speedup vs baseline: 24.0628x; 24.0628x over previous
import functools
import math

import jax
import jax.numpy as jnp
from jax.experimental import pallas as pl

B, N, DG, DF, K = 8, 1024, 256, 256, 16
BLK = 256
NEG = float("-inf")


def _attn_kernel(xyz_blk_ref, xyz_ref, cat_blk_ref, cat_ref, feat_ref,
                 wq_ref, bq_ref, wk_ref, out_ref):
    x_blk = xyz_blk_ref[0]
    x_all = xyz_ref[0]
    inner = jax.lax.dot_general(
        x_blk, x_all, (((1,), (1,)), ((), ())),
        preferred_element_type=jnp.float32)
    xxb = jnp.sum(x_blk * x_blk, axis=1, keepdims=True)
    xxa = jnp.sum(x_all * x_all, axis=1)[None, :]
    dist = 2.0 * inner - xxb - xxa

    work = dist
    thr = None
    for _ in range(K):
        thr = jnp.max(work, axis=1, keepdims=True)
        work = jnp.where(work >= thr, NEG, work)
    mask = dist >= thr

    c_blk = cat_blk_ref[0]
    q = jax.lax.dot_general(
        c_blk, wq_ref[...], (((1,), (1,)), ((), ())),
        preferred_element_type=jnp.float32) + bq_ref[...]
    qk = jnp.dot(q, wk_ref[...], preferred_element_type=jnp.float32)
    s = jax.lax.dot_general(
        qk, cat_ref[0], (((1,), (1,)), ((), ())),
        preferred_element_type=jnp.float32) * (1.0 / math.sqrt(DG))
    s = jnp.where(mask, s, NEG)
    m = jnp.max(s, axis=1, keepdims=True)
    p = jnp.where(mask, jnp.exp(s - m), 0.0)
    a = p / jnp.sum(p, axis=1, keepdims=True)
    out_ref[0] = jax.lax.dot_general(
        feat_ref[0], a, (((1,), (1,)), ((), ())),
        preferred_element_type=jnp.float32)


@jax.jit
def kernel(fp4_xyz, fp4_features, concatenate_features, Wq, bq, Wk, bk):
    del bk
    nb = N // BLK
    grid = (B, nb)
    out = pl.pallas_call(
        _attn_kernel,
        grid=grid,
        in_specs=[
            pl.BlockSpec((1, BLK, 3), lambda b, n: (b, n, 0)),
            pl.BlockSpec((1, N, 3), lambda b, n: (b, 0, 0)),
            pl.BlockSpec((1, BLK, DG), lambda b, n: (b, n, 0)),
            pl.BlockSpec((1, N, DG), lambda b, n: (b, 0, 0)),
            pl.BlockSpec((1, DF, N), lambda b, n: (b, 0, 0)),
            pl.BlockSpec((DG, DG), lambda b, n: (0, 0)),
            pl.BlockSpec((1, DG), lambda b, n: (0, 0)),
            pl.BlockSpec((DG, DG), lambda b, n: (0, 0)),
        ],
        out_specs=pl.BlockSpec((1, DF, BLK), lambda b, n: (b, 0, n)),
        out_shape=jax.ShapeDtypeStruct((B, DF, N), jnp.float32),
    )(fp4_xyz, fp4_xyz, concatenate_features, concatenate_features,
      fp4_features, Wq, bq.reshape(1, DG), Wk)
    return out
